# baseline (speedup 1.0000x reference)
import functools

import jax
import jax.numpy as jnp
from jax import lax
from jax.experimental import pallas as pl
from jax.experimental.pallas import tpu as pltpu
from jax.experimental.pallas import tpu_sc as plsc

_B = 32
_NPOS = 64512
_ROWS = 32

_LVL_S32 = (32, 1024, 32.0, 0,
            ((91.7364, 144.9949), (137.5189, 178.4791), (194.4429, 250.7985)),
            1, 8)
_LVL_S16 = (64, 4096, 16.0, 3072,
            ((45.0668, 101.4673), (63.0952, 113.5382), (81.3909, 134.4554)),
            1, 32)
_LVL_S8 = (128, 16384, 8.0, 15360,
           ((18.7807, 33.4659), (28.8912, 61.7536), (48.6849, 68.3897)),
           4, 32)


def _pipeline(raws, levels, out,
              in_bufs, stages, sem_in0, sem_in1, sem_out0, sem_out1):
    b = lax.axis_index("s") * 2 + lax.axis_index("c")
    g = b >> 3
    r = b & 7
    iota_f = lax.iota(jnp.int32, 16).astype(jnp.float32)

    sems_in = (sem_in0, sem_in1)
    sems_out = (sem_out0, sem_out1)

    groups = []
    for li, (W, HW, stride, pos_off, anchors, n_chunks, rows) in enumerate(
            levels):
        for a in range(3):
            for k in range(n_chunks):
                groups.append((li, W, HW, stride, pos_off, anchors[a], a, k,
                               rows))
    ng = len(groups)

    def start_in(gi):
        li, W, HW, stride, pos_off, anc, a, k, rows = groups[gi]
        par = gi % 2
        return pltpu.async_copy(
            raws[li].at[b, pl.ds(a * 6, 6), pl.ds(k * rows, rows)],
            in_bufs.at[par, pl.ds(0, 6), pl.ds(0, rows)],
            sems_in[par])

    def start_out(gi):
        li, W, HW, stride, pos_off, anc, a, k, rows = groups[gi]
        par = gi % 2
        pc0 = (pos_off + a * HW + k * rows * 128) >> 7
        return [
            pltpu.async_copy(
                stages.at[par, f, pl.ds(0, rows)],
                out.at[f, g, pl.ds(pc0, rows), pl.ds(r, 1)],
                sems_out[par])
            for f in range(6)
        ]

    def compute(gi):
        li, W, HW, stride, pos_off, anc, a, k, rows = groups[gi]
        par = gi % 2
        aw, ah = anc
        lw = W.bit_length() - 1
        rpy = 128 >> lw
        p0y = (k * rows * 128) >> lw
        xvs = [(iota_f + float((u * 16) & (W - 1))) * stride
               for u in range(8)]

        @plsc.parallel_loop(0, rows, 1, unroll=1)
        def body(row):
            ybase = p0y + row * rpy
            for u in range(8):
                xr = u * 16
                yf = (ybase + ((u * 16) >> lw)).astype(jnp.float32)
                v0 = in_bufs[par, 0, row, pl.ds(xr, 16)]
                v1 = in_bufs[par, 1, row, pl.ds(xr, 16)]
                v2 = in_bufs[par, 2, row, pl.ds(xr, 16)]
                v3 = in_bufs[par, 3, row, pl.ds(xr, 16)]
                v4 = in_bufs[par, 4, row, pl.ds(xr, 16)]
                v5 = in_bufs[par, 5, row, pl.ds(xr, 16)]
                a0 = 1.0 + jnp.exp(-v0)
                a1 = 1.0 + jnp.exp(-v1)
                a4 = 1.0 + jnp.exp(-v4)
                a5 = 1.0 + jnp.exp(-v5)
                p01 = a0 * a1
                p45 = a4 * a5
                rinv = 1.0 / (p01 * p45)
                r01 = rinv * p45
                r45 = rinv * p01
                s0 = r01 * a1
                s1 = r01 * a0
                s4 = r45 * a5
                s5 = r45 * a4
                stages[par, 0, row, 0, pl.ds(xr, 16)] = s0 * stride + xvs[u]
                stages[par, 1, row, 0, pl.ds(xr, 16)] = (s1 + yf) * stride
                stages[par, 2, row, 0, pl.ds(xr, 16)] = jnp.exp(v2) * aw
                stages[par, 3, row, 0, pl.ds(xr, 16)] = jnp.exp(v3) * ah
                stages[par, 4, row, 0, pl.ds(xr, 16)] = s4 * 360.0 - 180.0
                stages[par, 5, row, 0, pl.ds(xr, 16)] = s5

    handles_out = [None] * ng
    h_in_next = start_in(0)
    for gi in range(ng):
        h_in_cur = h_in_next
        if gi + 1 < ng:
            h_in_next = start_in(gi + 1)
        h_in_cur.wait()
        if gi >= 2:
            for h in handles_out[gi - 2]:
                h.wait()
        compute(gi)
        handles_out[gi] = start_out(gi)
    for gi in (ng - 2, ng - 1):
        for h in handles_out[gi]:
            h.wait()


def _body_s8(rs8, out, *scratch):
    _pipeline((rs8,), (_LVL_S8,), out, *scratch)


def _body_small(rs32, rs16, out, *scratch):
    _pipeline((rs32, rs16), (_LVL_S32, _LVL_S16), out, *scratch)


_SCRATCH = [
    pltpu.VMEM((2, 6, _ROWS, 128), jnp.float32),
    pltpu.VMEM((2, 6, _ROWS, 1, 128), jnp.float32),
    pltpu.SemaphoreType.DMA,
    pltpu.SemaphoreType.DMA,
    pltpu.SemaphoreType.DMA,
    pltpu.SemaphoreType.DMA,
]


@jax.jit
def _decode(raw_s32, raw_s16, raw_s8):
    mesh = plsc.VectorSubcoreMesh(core_axis_name="c", subcore_axis_name="s")
    params = pltpu.CompilerParams(needs_layout_passes=False)
    rs32 = raw_s32.reshape(_B, 18, 8, 128)
    rs16 = raw_s16.reshape(_B, 18, 32, 128)
    rs8 = raw_s8.reshape(_B, 18, 128, 128)

    call_s8 = functools.partial(
        pl.kernel,
        out_type=jax.ShapeDtypeStruct((6, _B // 8, _NPOS // 128, 8, 128),
                                      jnp.float32),
        mesh=mesh, scratch_types=_SCRATCH, compiler_params=params,
    )(_body_s8)
    buf = call_s8(rs8)

    ref = jax.new_ref(buf)
    call_small = functools.partial(
        pl.kernel, out_type=(),
        mesh=mesh, scratch_types=_SCRATCH, compiler_params=params,
    )(_body_small)
    call_small(rs32, rs16, ref)
    return ref[...]


def kernel(raw_s32, raw_s16, raw_s8):
    buf = _decode(raw_s32, raw_s16, raw_s8)
    return buf.transpose(1, 3, 2, 4, 0).reshape(_B, _NPOS, 6)


# speedup vs baseline: 1.1061x; 1.1061x over previous
import functools

import jax
import jax.numpy as jnp
from jax import lax
from jax.experimental import pallas as pl
from jax.experimental.pallas import tpu as pltpu
from jax.experimental.pallas import tpu_sc as plsc

_B = 32
_NPOS = 64512
_ROWS = 32

_LVL_S32 = (32, 1024, 32.0, 0,
            ((91.7364, 144.9949), (137.5189, 178.4791), (194.4429, 250.7985)),
            1, 8)
_LVL_S16 = (64, 4096, 16.0, 3072,
            ((45.0668, 101.4673), (63.0952, 113.5382), (81.3909, 134.4554)),
            1, 32)
_LVL_S8 = (128, 16384, 8.0, 15360,
           ((18.7807, 33.4659), (28.8912, 61.7536), (48.6849, 68.3897)),
           4, 32)


def _pipeline(raws, levels, out,
              in_bufs, stages, sem_in0, sem_in1, sem_out0, sem_out1):
    b = lax.axis_index("s") * 2 + lax.axis_index("c")
    g = b >> 3
    r = b & 7
    iota_f = lax.iota(jnp.int32, 16).astype(jnp.float32)

    sems_in = (sem_in0, sem_in1)
    sems_out = (sem_out0, sem_out1)

    groups = []
    for li, (W, HW, stride, pos_off, anchors, n_chunks, rows) in enumerate(
            levels):
        for a in range(3):
            for k in range(n_chunks):
                groups.append((li, W, HW, stride, pos_off, anchors[a], a, k,
                               rows))
    ng = len(groups)

    def start_in(gi):
        li, W, HW, stride, pos_off, anc, a, k, rows = groups[gi]
        par = gi % 2
        return pltpu.async_copy(
            raws[li].at[b, pl.ds(a * 6, 6), pl.ds(k * rows, rows)],
            in_bufs.at[par, pl.ds(0, 6), pl.ds(0, rows)],
            sems_in[par])

    def start_out(gi):
        li, W, HW, stride, pos_off, anc, a, k, rows = groups[gi]
        par = gi % 2
        pc0 = (pos_off + a * HW + k * rows * 128) >> 7
        return [
            pltpu.async_copy(
                stages.at[par, f, pl.ds(0, rows)],
                out.at[f, g, pl.ds(pc0, rows), pl.ds(r, 1)],
                sems_out[par])
            for f in range(6)
        ]

    def compute(gi):
        li, W, HW, stride, pos_off, anc, a, k, rows = groups[gi]
        par = gi % 2
        aw, ah = anc
        lw = W.bit_length() - 1
        p0 = k * rows * 128

        @plsc.parallel_loop(0, rows * 8, 1, unroll=4)
        def body(i):
            yv = i >> 3
            xr = (i & 7) * 16
            pos = p0 + i * 16
            x0 = (pos & (W - 1)).astype(jnp.float32)
            y0 = (pos >> lw).astype(jnp.float32)
            xv = iota_f + x0
            v0 = in_bufs[par, 0, yv, pl.ds(xr, 16)]
            v1 = in_bufs[par, 1, yv, pl.ds(xr, 16)]
            v2 = in_bufs[par, 2, yv, pl.ds(xr, 16)]
            v3 = in_bufs[par, 3, yv, pl.ds(xr, 16)]
            v4 = in_bufs[par, 4, yv, pl.ds(xr, 16)]
            v5 = in_bufs[par, 5, yv, pl.ds(xr, 16)]
            a0 = 1.0 + jnp.exp(-v0)
            a1 = 1.0 + jnp.exp(-v1)
            a4 = 1.0 + jnp.exp(-v4)
            a5 = 1.0 + jnp.exp(-v5)
            p01 = a0 * a1
            p45 = a4 * a5
            rinv = 1.0 / (p01 * p45)
            r01 = rinv * p45
            r45 = rinv * p01
            s0 = r01 * a1
            s1 = r01 * a0
            s4 = r45 * a5
            s5 = r45 * a4
            stages[par, 0, yv, 0, pl.ds(xr, 16)] = (s0 + xv) * stride
            stages[par, 1, yv, 0, pl.ds(xr, 16)] = (s1 + y0) * stride
            stages[par, 2, yv, 0, pl.ds(xr, 16)] = jnp.exp(v2) * aw
            stages[par, 3, yv, 0, pl.ds(xr, 16)] = jnp.exp(v3) * ah
            stages[par, 4, yv, 0, pl.ds(xr, 16)] = s4 * 360.0 - 180.0
            stages[par, 5, yv, 0, pl.ds(xr, 16)] = s5

    handles_out = [None] * ng
    h_in_next = start_in(0)
    for gi in range(ng):
        h_in_cur = h_in_next
        if gi + 1 < ng:
            h_in_next = start_in(gi + 1)
        h_in_cur.wait()
        if gi >= 2:
            for h in handles_out[gi - 2]:
                h.wait()
        compute(gi)
        handles_out[gi] = start_out(gi)
    for gi in (ng - 2, ng - 1):
        for h in handles_out[gi]:
            h.wait()


def _body_s8(rs8, out, *scratch):
    _pipeline((rs8,), (_LVL_S8,), out, *scratch)


def _body_small(rs32, rs16, out, *scratch):
    _pipeline((rs32, rs16), (_LVL_S32, _LVL_S16), out, *scratch)


_SCRATCH = [
    pltpu.VMEM((2, 6, _ROWS, 128), jnp.float32),
    pltpu.VMEM((2, 6, _ROWS, 1, 128), jnp.float32),
    pltpu.SemaphoreType.DMA,
    pltpu.SemaphoreType.DMA,
    pltpu.SemaphoreType.DMA,
    pltpu.SemaphoreType.DMA,
]


@jax.jit
def _decode(raw_s32, raw_s16, raw_s8):
    mesh = plsc.VectorSubcoreMesh(core_axis_name="c", subcore_axis_name="s")
    params = pltpu.CompilerParams(needs_layout_passes=False)
    rs32 = raw_s32.reshape(_B, 18, 8, 128)
    rs16 = raw_s16.reshape(_B, 18, 32, 128)
    rs8 = raw_s8.reshape(_B, 18, 128, 128)

    call_s8 = functools.partial(
        pl.kernel,
        out_type=jax.ShapeDtypeStruct((6, _B // 8, _NPOS // 128, 8, 128),
                                      jnp.float32),
        mesh=mesh, scratch_types=_SCRATCH, compiler_params=params,
    )(_body_s8)
    buf = call_s8(rs8)

    ref = jax.new_ref(buf)
    call_small = functools.partial(
        pl.kernel, out_type=(),
        mesh=mesh, scratch_types=_SCRATCH, compiler_params=params,
    )(_body_small)
    call_small(rs32, rs16, ref)
    return ref[...]


def kernel(raw_s32, raw_s16, raw_s8):
    buf = _decode(raw_s32, raw_s16, raw_s8)
    return buf.transpose(1, 3, 2, 4, 0).reshape(_B, _NPOS, 6)


# speedup vs baseline: 1.1117x; 1.0051x over previous
import functools

import jax
import jax.numpy as jnp
from jax import lax
from jax.experimental import pallas as pl
from jax.experimental.pallas import tpu as pltpu
from jax.experimental.pallas import tpu_sc as plsc

_B = 32
_NPOS = 64512
_ROWS = 32

_LVL_S32 = (32, 1024, 32.0, 0,
            ((91.7364, 144.9949), (137.5189, 178.4791), (194.4429, 250.7985)),
            1, 8)
_LVL_S16 = (64, 4096, 16.0, 3072,
            ((45.0668, 101.4673), (63.0952, 113.5382), (81.3909, 134.4554)),
            1, 32)
_LVL_S8 = (128, 16384, 8.0, 15360,
           ((18.7807, 33.4659), (28.8912, 61.7536), (48.6849, 68.3897)),
           4, 32)


def _pipeline(raws, levels, out,
              in_bufs, stages, sem_in0, sem_in1, sem_out0, sem_out1):
    b = lax.axis_index("s") * 2 + lax.axis_index("c")
    g = b >> 3
    r = b & 7
    iota_f = lax.iota(jnp.int32, 16).astype(jnp.float32)

    sems_in = (sem_in0, sem_in1)
    sems_out = (sem_out0, sem_out1)

    groups = []
    for li, (W, HW, stride, pos_off, anchors, n_chunks, rows) in enumerate(
            levels):
        for a in range(3):
            for k in range(n_chunks):
                groups.append((li, W, HW, stride, pos_off, anchors[a], a, k,
                               rows))
    ng = len(groups)

    def start_in(gi):
        li, W, HW, stride, pos_off, anc, a, k, rows = groups[gi]
        par = gi % 2
        return pltpu.async_copy(
            raws[li].at[b, pl.ds(a * 6, 6), pl.ds(k * rows, rows)],
            in_bufs.at[par, pl.ds(0, 6), pl.ds(0, rows)],
            sems_in[par])

    def start_out(gi):
        li, W, HW, stride, pos_off, anc, a, k, rows = groups[gi]
        par = gi % 2
        pc0 = (pos_off + a * HW + k * rows * 128) >> 7
        return [
            pltpu.async_copy(
                stages.at[par, f, pl.ds(0, rows)],
                out.at[f, g, pl.ds(pc0, rows), pl.ds(r, 1)],
                sems_out[par])
            for f in range(6)
        ]

    def compute(gi):
        li, W, HW, stride, pos_off, anc, a, k, rows = groups[gi]
        par = gi % 2
        aw, ah = anc
        lw = W.bit_length() - 1
        p0 = k * rows * 128

        @plsc.parallel_loop(0, rows * 8, 1, unroll=2)
        def body(i):
            yv = i >> 3
            xr = (i & 7) * 16
            pos = p0 + i * 16
            x0 = (pos & (W - 1)).astype(jnp.float32)
            y0 = (pos >> lw).astype(jnp.float32)
            xv = iota_f + x0
            v0 = in_bufs[par, 0, yv, pl.ds(xr, 16)]
            v1 = in_bufs[par, 1, yv, pl.ds(xr, 16)]
            v2 = in_bufs[par, 2, yv, pl.ds(xr, 16)]
            v3 = in_bufs[par, 3, yv, pl.ds(xr, 16)]
            v4 = in_bufs[par, 4, yv, pl.ds(xr, 16)]
            v5 = in_bufs[par, 5, yv, pl.ds(xr, 16)]
            a0 = 1.0 + jnp.exp(-v0)
            a1 = 1.0 + jnp.exp(-v1)
            a4 = 1.0 + jnp.exp(-v4)
            a5 = 1.0 + jnp.exp(-v5)
            p01 = a0 * a1
            p45 = a4 * a5
            rinv = 1.0 / (p01 * p45)
            r01 = rinv * p45
            r45 = rinv * p01
            s0 = r01 * a1
            s1 = r01 * a0
            s4 = r45 * a5
            s5 = r45 * a4
            stages[par, 0, yv, 0, pl.ds(xr, 16)] = (s0 + xv) * stride
            stages[par, 1, yv, 0, pl.ds(xr, 16)] = (s1 + y0) * stride
            stages[par, 2, yv, 0, pl.ds(xr, 16)] = jnp.exp(v2) * aw
            stages[par, 3, yv, 0, pl.ds(xr, 16)] = jnp.exp(v3) * ah
            stages[par, 4, yv, 0, pl.ds(xr, 16)] = s4 * 360.0 - 180.0
            stages[par, 5, yv, 0, pl.ds(xr, 16)] = s5

    handles_out = [None] * ng
    h_in_next = start_in(0)
    for gi in range(ng):
        h_in_cur = h_in_next
        if gi + 1 < ng:
            h_in_next = start_in(gi + 1)
        h_in_cur.wait()
        if gi >= 2:
            for h in handles_out[gi - 2]:
                h.wait()
        compute(gi)
        handles_out[gi] = start_out(gi)
    for gi in (ng - 2, ng - 1):
        for h in handles_out[gi]:
            h.wait()


def _body_s8(rs8, out, *scratch):
    _pipeline((rs8,), (_LVL_S8,), out, *scratch)


def _body_small(rs32, rs16, out, *scratch):
    _pipeline((rs32, rs16), (_LVL_S32, _LVL_S16), out, *scratch)


_SCRATCH = [
    pltpu.VMEM((2, 6, _ROWS, 128), jnp.float32),
    pltpu.VMEM((2, 6, _ROWS, 1, 128), jnp.float32),
    pltpu.SemaphoreType.DMA,
    pltpu.SemaphoreType.DMA,
    pltpu.SemaphoreType.DMA,
    pltpu.SemaphoreType.DMA,
]


@jax.jit
def _decode(raw_s32, raw_s16, raw_s8):
    mesh = plsc.VectorSubcoreMesh(core_axis_name="c", subcore_axis_name="s")
    params = pltpu.CompilerParams(needs_layout_passes=False,
                                  skip_device_barrier=True)
    rs32 = raw_s32.reshape(_B, 18, 8, 128)
    rs16 = raw_s16.reshape(_B, 18, 32, 128)
    rs8 = raw_s8.reshape(_B, 18, 128, 128)

    call_s8 = functools.partial(
        pl.kernel,
        out_type=jax.ShapeDtypeStruct((6, _B // 8, _NPOS // 128, 8, 128),
                                      jnp.float32),
        mesh=mesh, scratch_types=_SCRATCH, compiler_params=params,
    )(_body_s8)
    buf = call_s8(rs8)

    ref = jax.new_ref(buf)
    call_small = functools.partial(
        pl.kernel, out_type=(),
        mesh=mesh, scratch_types=_SCRATCH, compiler_params=params,
    )(_body_small)
    call_small(rs32, rs16, ref)
    return ref[...]


def kernel(raw_s32, raw_s16, raw_s8):
    buf = _decode(raw_s32, raw_s16, raw_s8)
    return buf.transpose(1, 3, 2, 4, 0).reshape(_B, _NPOS, 6)


# speedup vs baseline: 1.1277x; 1.0144x over previous
import functools

import jax
import jax.numpy as jnp
from jax import lax
from jax.experimental import pallas as pl
from jax.experimental.pallas import tpu as pltpu
from jax.experimental.pallas import tpu_sc as plsc

_B = 32
_NPOS = 64512
_ROWS = 32

_LVL_S32 = (32, 1024, 32.0, 0,
            ((91.7364, 144.9949), (137.5189, 178.4791), (194.4429, 250.7985)),
            1, 8)
_LVL_S16 = (64, 4096, 16.0, 3072,
            ((45.0668, 101.4673), (63.0952, 113.5382), (81.3909, 134.4554)),
            1, 32)
_LVL_S8 = (128, 16384, 8.0, 15360,
           ((18.7807, 33.4659), (28.8912, 61.7536), (48.6849, 68.3897)),
           4, 32)


def _pipeline(raws, levels, out,
              in_bufs, stages, sem_in0, sem_in1, sem_out0, sem_out1):
    b = lax.axis_index("s") * 2 + lax.axis_index("c")
    g = b >> 3
    r = b & 7
    iota_f = lax.iota(jnp.int32, 16).astype(jnp.float32)

    sems_in = (sem_in0, sem_in1)
    sems_out = (sem_out0, sem_out1)

    groups = []
    for li, (W, HW, stride, pos_off, anchors, n_chunks, rows) in enumerate(
            levels):
        for a in range(3):
            for k in range(n_chunks):
                groups.append((li, W, HW, stride, pos_off, anchors[a], a, k,
                               rows))
    ng = len(groups)

    def start_in(gi):
        li, W, HW, stride, pos_off, anc, a, k, rows = groups[gi]
        par = gi % 2
        return pltpu.async_copy(
            raws[li].at[b, pl.ds(a * 6, 6), pl.ds(k * rows, rows)],
            in_bufs.at[par, pl.ds(0, 6), pl.ds(0, rows)],
            sems_in[par])

    def start_out(gi):
        li, W, HW, stride, pos_off, anc, a, k, rows = groups[gi]
        par = gi % 2
        pc0 = (pos_off + a * HW + k * rows * 128) >> 7
        return [
            pltpu.async_copy(
                stages.at[par, pl.ds(0, 6), pl.ds(0, rows)],
                out.at[pl.ds(0, 6), g, pl.ds(pc0, rows), pl.ds(r, 1)],
                sems_out[par])
        ]

    def compute(gi):
        li, W, HW, stride, pos_off, anc, a, k, rows = groups[gi]
        par = gi % 2
        aw, ah = anc
        lw = W.bit_length() - 1
        p0 = k * rows * 128

        @plsc.parallel_loop(0, rows * 8, 1, unroll=2)
        def body(i):
            yv = i >> 3
            xr = (i & 7) * 16
            pos = p0 + i * 16
            x0 = (pos & (W - 1)).astype(jnp.float32)
            y0 = (pos >> lw).astype(jnp.float32)
            xv = iota_f + x0
            v0 = in_bufs[par, 0, yv, pl.ds(xr, 16)]
            v1 = in_bufs[par, 1, yv, pl.ds(xr, 16)]
            v2 = in_bufs[par, 2, yv, pl.ds(xr, 16)]
            v3 = in_bufs[par, 3, yv, pl.ds(xr, 16)]
            v4 = in_bufs[par, 4, yv, pl.ds(xr, 16)]
            v5 = in_bufs[par, 5, yv, pl.ds(xr, 16)]
            a0 = 1.0 + jnp.exp(-v0)
            a1 = 1.0 + jnp.exp(-v1)
            a4 = 1.0 + jnp.exp(-v4)
            a5 = 1.0 + jnp.exp(-v5)
            p01 = a0 * a1
            p45 = a4 * a5
            rinv = 1.0 / (p01 * p45)
            r01 = rinv * p45
            r45 = rinv * p01
            s0 = r01 * a1
            s1 = r01 * a0
            s4 = r45 * a5
            s5 = r45 * a4
            stages[par, 0, yv, 0, pl.ds(xr, 16)] = (s0 + xv) * stride
            stages[par, 1, yv, 0, pl.ds(xr, 16)] = (s1 + y0) * stride
            stages[par, 2, yv, 0, pl.ds(xr, 16)] = jnp.exp(v2) * aw
            stages[par, 3, yv, 0, pl.ds(xr, 16)] = jnp.exp(v3) * ah
            stages[par, 4, yv, 0, pl.ds(xr, 16)] = s4 * 360.0 - 180.0
            stages[par, 5, yv, 0, pl.ds(xr, 16)] = s5

    handles_out = [None] * ng
    h_in_next = start_in(0)
    for gi in range(ng):
        h_in_cur = h_in_next
        if gi + 1 < ng:
            h_in_next = start_in(gi + 1)
        h_in_cur.wait()
        if gi >= 2:
            for h in handles_out[gi - 2]:
                h.wait()
        compute(gi)
        handles_out[gi] = start_out(gi)
    for gi in (ng - 2, ng - 1):
        for h in handles_out[gi]:
            h.wait()


def _body_s8(rs8, out, *scratch):
    _pipeline((rs8,), (_LVL_S8,), out, *scratch)


def _body_small(rs32, rs16, out, *scratch):
    _pipeline((rs32, rs16), (_LVL_S32, _LVL_S16), out, *scratch)


_SCRATCH = [
    pltpu.VMEM((2, 6, _ROWS, 128), jnp.float32),
    pltpu.VMEM((2, 6, _ROWS, 1, 128), jnp.float32),
    pltpu.SemaphoreType.DMA,
    pltpu.SemaphoreType.DMA,
    pltpu.SemaphoreType.DMA,
    pltpu.SemaphoreType.DMA,
]


@jax.jit
def _decode(raw_s32, raw_s16, raw_s8):
    mesh = plsc.VectorSubcoreMesh(core_axis_name="c", subcore_axis_name="s")
    params = pltpu.CompilerParams(needs_layout_passes=False,
                                  skip_device_barrier=True)
    rs32 = raw_s32.reshape(_B, 18, 8, 128)
    rs16 = raw_s16.reshape(_B, 18, 32, 128)
    rs8 = raw_s8.reshape(_B, 18, 128, 128)

    call_s8 = functools.partial(
        pl.kernel,
        out_type=jax.ShapeDtypeStruct((6, _B // 8, _NPOS // 128, 8, 128),
                                      jnp.float32),
        mesh=mesh, scratch_types=_SCRATCH, compiler_params=params,
    )(_body_s8)
    buf = call_s8(rs8)

    ref = jax.new_ref(buf)
    call_small = functools.partial(
        pl.kernel, out_type=(),
        mesh=mesh, scratch_types=_SCRATCH, compiler_params=params,
    )(_body_small)
    call_small(rs32, rs16, ref)
    return ref[...]


def kernel(raw_s32, raw_s16, raw_s8):
    buf = _decode(raw_s32, raw_s16, raw_s8)
    return buf.transpose(1, 3, 2, 4, 0).reshape(_B, _NPOS, 6)
